# baseline (speedup 1.0000x reference)
import jax
import jax.numpy as jnp
from jax.experimental import pallas as pl
from jax.experimental.pallas import tpu as pltpu

N = 4096
D = 256
NLAYERS = 5
ALPHA = 0.2
BI = 512
CH = 512
NBLK = N // BI
NCH = N // CH
GRID_J = NBLK + 1
NEG = -3e38


def _gat_body(feat_ref, adj_ref, w_ref, a1_ref, a2_ref, out_ref,
              xbuf, whbbuf, f1buf, f2rbuf, maskbuf):
    l = pl.program_id(0)
    j = pl.program_id(1)

    @pl.when(j == 0)
    def _prologue():
        @pl.when(l == 0)
        def _():
            xbuf[...] = feat_ref[...]
        x = xbuf[...]
        wh = jnp.dot(x, w_ref[0], preferred_element_type=jnp.float32)
        whbbuf[...] = wh.astype(jnp.bfloat16)
        f1buf[...] = jnp.sum(wh * a1_ref[0], axis=1, keepdims=True)
        f2 = jnp.sum(wh * a2_ref[0], axis=1, keepdims=True)
        f2rbuf[...] = jnp.transpose(f2)

    @pl.when(j > 0)
    def _attention():
        r = (j - 1) * BI
        f1 = f1buf[pl.ds(r, BI), :]

        def _chunks(get_mn):
            acc = jnp.zeros((BI, D), jnp.float32)
            den = jnp.zeros((BI, 1), jnp.float32)
            for c in range(NCH):
                cs = c * CH
                s = f1 + f2rbuf[:, cs:cs + CH]
                e = jnp.maximum(s, ALPHA * s)
                p = jnp.exp(e + get_mn(c, cs))
                den = den + jnp.sum(p, axis=1, keepdims=True)
                acc = acc + jnp.dot(p.astype(jnp.bfloat16),
                                    whbbuf[pl.ds(cs, CH), :],
                                    preferred_element_type=jnp.float32)
            o = acc * (1.0 / den)

            @pl.when(l < NLAYERS - 1)
            def _():
                xbuf[pl.ds(r, BI), :] = jnp.maximum(o, 0.0)

            @pl.when(l == NLAYERS - 1)
            def _():
                out_ref[...] = o

        @pl.when((l == 0) | (j == NBLK))
        def _():
            def get_mn(c, cs):
                mn = jnp.where(adj_ref[:, cs:cs + CH] > 0.0, 0.0, NEG)

                @pl.when((l == 0) & (j < NBLK))
                def _():
                    maskbuf[pl.ds(r, BI), cs:cs + CH] = mn.astype(jnp.bfloat16)

                return mn

            _chunks(get_mn)

        @pl.when((l > 0) & (j < NBLK))
        def _():
            _chunks(lambda c, cs:
                    maskbuf[pl.ds(r, BI), cs:cs + CH].astype(jnp.float32))


def _logsoftmax_body(x_ref, out_ref):
    x = x_ref[...]
    m0 = jnp.max(x, axis=0, keepdims=True)
    lse = jnp.log(jnp.sum(jnp.exp(x - m0), axis=0, keepdims=True)) + m0
    out_ref[...] = x - lse


def kernel(features, adj_matrix, W1, a1, W2, a2, W3, a3, W4, a4, W5, a5):
    Ws = jnp.stack([W1, W2, W3, W4, W5])
    als = [a1, a2, a3, a4, a5]
    a1s = jnp.stack([a[:D].reshape(1, D) for a in als])
    a2s = jnp.stack([a[D:].reshape(1, D) for a in als])

    x = pl.pallas_call(
        _gat_body,
        grid=(NLAYERS, GRID_J),
        in_specs=[
            pl.BlockSpec((N, D), lambda l, j: (0, 0)),
            pl.BlockSpec(
                (BI, N),
                lambda l, j: (jnp.where(l == 0, jnp.maximum(j - 1, 0), NBLK - 1), 0),
            ),
            pl.BlockSpec((1, D, D), lambda l, j: (l, 0, 0)),
            pl.BlockSpec((1, 1, D), lambda l, j: (l, 0, 0)),
            pl.BlockSpec((1, 1, D), lambda l, j: (l, 0, 0)),
        ],
        out_specs=pl.BlockSpec(
            (BI, D),
            lambda l, j: (jnp.where(l == NLAYERS - 1, jnp.maximum(j - 1, 0), 0), 0),
        ),
        out_shape=jax.ShapeDtypeStruct((N, D), jnp.float32),
        compiler_params=pltpu.CompilerParams(
            vmem_limit_bytes=128 * 1024 * 1024,
        ),
        scratch_shapes=[
            pltpu.VMEM((N, D), jnp.float32),
            pltpu.VMEM((N, D), jnp.bfloat16),
            pltpu.VMEM((N, 1), jnp.float32),
            pltpu.VMEM((1, N), jnp.float32),
            pltpu.VMEM((N - BI, N), jnp.bfloat16),
        ],
    )(features, adj_matrix, Ws, a1s, a2s)

    out = pl.pallas_call(
        _logsoftmax_body,
        out_shape=jax.ShapeDtypeStruct((N, D), jnp.float32),
    )(x)
    return out


# speedup vs baseline: 1.0894x; 1.0894x over previous
import jax
import jax.numpy as jnp
from jax.experimental import pallas as pl
from jax.experimental.pallas import tpu as pltpu

N = 4096
D = 256
NLAYERS = 5
ALPHA = 0.2
BI = 256
NBLK = N // BI
GRID_J = NBLK + 1
NEG = -3e38


def _gat_body(feat_ref, adj_ref, w_ref, a1_ref, a2_ref, out_ref,
              xbuf, whbuf, whbbuf, f2rbuf, maskbuf):
    l = pl.program_id(0)
    j = pl.program_id(1)

    @pl.when(j == 0)
    def _prologue():
        @pl.when(l == 0)
        def _():
            xbuf[...] = feat_ref[...]
        x = xbuf[...]
        wh = jnp.dot(x, w_ref[0], preferred_element_type=jnp.float32)
        whbuf[...] = wh
        whbbuf[...] = wh.astype(jnp.bfloat16)
        f2 = jnp.sum(wh * a2_ref[0], axis=1, keepdims=True)
        f2rbuf[...] = jnp.transpose(f2)

    @pl.when(j > 0)
    def _attention():
        r = (j - 1) * BI
        f1 = jnp.sum(whbuf[pl.ds(r, BI), :] * a1_ref[0], axis=1,
                     keepdims=True)
        s = f1 + f2rbuf[...]
        e = jnp.maximum(s, ALPHA * s)

        def _finish(p):
            recip = 1.0 / jnp.sum(p, axis=1, keepdims=True)
            o = jnp.dot(p.astype(jnp.bfloat16), whbbuf[...],
                        preferred_element_type=jnp.float32)
            o = o * recip

            @pl.when(l < NLAYERS - 1)
            def _():
                xbuf[pl.ds(r, BI), :] = jnp.maximum(o, 0.0)

            @pl.when(l == NLAYERS - 1)
            def _():
                out_ref[...] = o

        @pl.when((l == 0) | (j == NBLK))
        def _():
            mn = jnp.where(adj_ref[...] > 0.0, 0.0, NEG)

            @pl.when((l == 0) & (j < NBLK))
            def _():
                maskbuf[pl.ds(r, BI), :] = mn.astype(jnp.bfloat16)

            _finish(jnp.exp(e + mn))

        @pl.when((l > 0) & (j < NBLK))
        def _():
            mn = maskbuf[pl.ds(r, BI), :].astype(jnp.float32)
            _finish(jnp.exp(e + mn))


def _logsoftmax_body(x_ref, out_ref):
    x = x_ref[...]
    m0 = jnp.max(x, axis=0, keepdims=True)
    lse = jnp.log(jnp.sum(jnp.exp(x - m0), axis=0, keepdims=True)) + m0
    out_ref[...] = x - lse


def kernel(features, adj_matrix, W1, a1, W2, a2, W3, a3, W4, a4, W5, a5):
    Ws = jnp.stack([W1, W2, W3, W4, W5])
    als = [a1, a2, a3, a4, a5]
    a1s = jnp.stack([a[:D].reshape(1, D) for a in als])
    a2s = jnp.stack([a[D:].reshape(1, D) for a in als])

    x = pl.pallas_call(
        _gat_body,
        grid=(NLAYERS, GRID_J),
        in_specs=[
            pl.BlockSpec((N, D), lambda l, j: (0, 0)),
            pl.BlockSpec(
                (BI, N),
                lambda l, j: (jnp.where(l == 0, jnp.maximum(j - 1, 0), NBLK - 1), 0),
            ),
            pl.BlockSpec((1, D, D), lambda l, j: (l, 0, 0)),
            pl.BlockSpec((1, 1, D), lambda l, j: (l, 0, 0)),
            pl.BlockSpec((1, 1, D), lambda l, j: (l, 0, 0)),
        ],
        out_specs=pl.BlockSpec(
            (BI, D),
            lambda l, j: (jnp.where(l == NLAYERS - 1, jnp.maximum(j - 1, 0), 0), 0),
        ),
        out_shape=jax.ShapeDtypeStruct((N, D), jnp.float32),
        compiler_params=pltpu.CompilerParams(
            vmem_limit_bytes=128 * 1024 * 1024,
        ),
        scratch_shapes=[
            pltpu.VMEM((N, D), jnp.float32),
            pltpu.VMEM((N, D), jnp.float32),
            pltpu.VMEM((N, D), jnp.bfloat16),
            pltpu.VMEM((1, N), jnp.float32),
            pltpu.VMEM((N - BI, N), jnp.bfloat16),
        ],
    )(features, adj_matrix, Ws, a1s, a2s)

    out = pl.pallas_call(
        _logsoftmax_body,
        out_shape=jax.ShapeDtypeStruct((N, D), jnp.float32),
    )(x)
    return out
